# baseline (speedup 1.0000x reference)
import functools

import jax
import jax.numpy as jnp
from jax.experimental import pallas as pl
from jax.experimental.pallas import tpu as pltpu

E = 8
N = 2048
D = 1024
Z = 128
BN = 256


def _single_stage(h, Wr, br, Wc, bc, eps):
    p = jnp.dot(h, Wr, preferred_element_type=jnp.float32) + br
    mu = p[:, :Z]
    lv = p[:, Z:]
    z = mu + eps * jnp.exp(0.5 * lv)
    return h + jnp.dot(z, Wc, preferred_element_type=jnp.float32) + bc


def _expert_stage(h, wr_ref, br_ref, wc_cat_ref, bc_pad_ref, sel_ref, eps_ref):
    mus, elvs, tsums = [], [], []
    for e in range(E):
        p = jnp.dot(h, wr_ref[e], preferred_element_type=jnp.float32)
        p = p + br_ref[e : e + 1, :]
        mu = p[:, :Z]
        lv = p[:, Z:]
        elv = jnp.exp(lv)
        mus.append(mu)
        elvs.append(elv)
        tsums.append(mu * mu + (elv - lv))
    mu_all = jnp.concatenate(mus, axis=1)
    elv_all = jnp.concatenate(elvs, axis=1)
    tsum_all = jnp.concatenate(tsums, axis=1)
    kl_all = jnp.dot(tsum_all, sel_ref[...], preferred_element_type=jnp.float32)
    best_kl = kl_all[:, 0:1]
    widx = jnp.zeros_like(best_kl, dtype=jnp.int32)
    for e in range(1, E):
        kl_e = kl_all[:, e : e + 1]
        upd = kl_e > best_kl
        best_kl = jnp.where(upd, kl_e, best_kl)
        widx = jnp.where(upd, e, widx)
    eps_all = jnp.concatenate([eps_ref[e] for e in range(E)], axis=1)
    z_all = mu_all + eps_all * jnp.sqrt(elv_all)
    e_lane = jax.lax.broadcasted_iota(jnp.int32, (BN, E * Z), 1) >> 7
    z_cat = jnp.where(widx == e_lane, z_all, 0.0)
    delta = jnp.dot(z_cat, wc_cat_ref[...], preferred_element_type=jnp.float32)
    oh = jnp.where(
        widx == jax.lax.broadcasted_iota(jnp.int32, (BN, Z), 1), 1.0, 0.0
    )
    delta = delta + jnp.dot(oh, bc_pad_ref[...], preferred_element_type=jnp.float32)
    return h + delta


def _fused_kernel(
    x_ref,
    wre_ref, bre_ref, wce_ref, bce_ref,
    wr_ref, br_ref, wc_ref, bc_ref, sel_ref,
    wrd_ref, brd_ref, wcd_ref, bcd_ref,
    ee_ref, e1_ref, e2_ref, ed_ref,
    out_ref,
):
    h = x_ref[...]
    h = _single_stage(h, wre_ref[0], bre_ref[...], wce_ref[0], bce_ref[...], ee_ref[0])
    h = _expert_stage(h, wr_ref, br_ref, wc_ref, bc_ref, sel_ref, e1_ref)
    h = _expert_stage(h, wr_ref, br_ref, wc_ref, bc_ref, sel_ref, e2_ref)
    h = _single_stage(h, wrd_ref[0], brd_ref[...], wcd_ref[0], bcd_ref[...], ed_ref[0])
    out_ref[...] = h


def _const_spec(shape):
    nd = len(shape)
    return pl.BlockSpec(shape, lambda i: (0,) * nd)


@jax.jit
def kernel(
    x,
    W_rec_enc, b_rec_enc, W_comp_enc, b_comp_enc,
    W_rec, b_rec, W_comp, b_comp,
    W_rec_dec, b_rec_dec, W_comp_dec, b_comp_dec,
    eps_enc, eps_c1, eps_c2, eps_dec,
):
    wc_cat = W_comp.reshape(E * Z, D)
    bc_pad = jnp.pad(b_comp, ((0, Z - E), (0, 0)))
    sel = jnp.repeat(jnp.eye(E, dtype=jnp.float32), Z, axis=0)
    sel = jnp.pad(sel, ((0, 0), (0, Z - E)))
    grid = (N // BN,)
    return pl.pallas_call(
        _fused_kernel,
        grid=grid,
        in_specs=[
            pl.BlockSpec((BN, D), lambda i: (i, 0)),
            _const_spec((1, D, 2 * Z)),
            _const_spec((1, 2 * Z)),
            _const_spec((1, Z, D)),
            _const_spec((1, D)),
            _const_spec((E, D, 2 * Z)),
            _const_spec((E, 2 * Z)),
            _const_spec((E * Z, D)),
            _const_spec((Z, D)),
            _const_spec((E * Z, Z)),
            _const_spec((1, D, 2 * Z)),
            _const_spec((1, 2 * Z)),
            _const_spec((1, Z, D)),
            _const_spec((1, D)),
            pl.BlockSpec((1, BN, Z), lambda i: (0, i, 0)),
            pl.BlockSpec((E, BN, Z), lambda i: (0, i, 0)),
            pl.BlockSpec((E, BN, Z), lambda i: (0, i, 0)),
            pl.BlockSpec((1, BN, Z), lambda i: (0, i, 0)),
        ],
        out_specs=pl.BlockSpec((BN, D), lambda i: (i, 0)),
        out_shape=jax.ShapeDtypeStruct((N, D), jnp.float32),
        compiler_params=pltpu.CompilerParams(
            dimension_semantics=("arbitrary",),
        ),
    )(
        x,
        W_rec_enc, b_rec_enc, W_comp_enc, b_comp_enc,
        W_rec, b_rec, wc_cat, bc_pad, sel,
        W_rec_dec, b_rec_dec, W_comp_dec, b_comp_dec,
        eps_enc, eps_c1, eps_c2, eps_dec,
    )


# speedup vs baseline: 1.4911x; 1.4911x over previous
import functools

import jax
import jax.numpy as jnp
from jax.experimental import pallas as pl
from jax.experimental.pallas import tpu as pltpu

E = 8
N = 2048
D = 1024
Z = 128
BN = 256


def _single_stage(h, Wr, br, Wc, bc, eps):
    p = jnp.dot(h, Wr, preferred_element_type=jnp.float32) + br
    mu = p[:, :Z]
    lv = p[:, Z:]
    z = mu + eps * jnp.exp(0.5 * lv)
    return h + jnp.dot(z, Wc, preferred_element_type=jnp.float32) + bc


def _expert_stage(h, wr_ref, br_ref, wc_cat_ref, bc_pad_ref, eps_ref):
    best_kl = None
    for e in range(E):
        p = jnp.dot(h, wr_ref[e], preferred_element_type=jnp.float32)
        p = p + br_ref[e : e + 1, :]
        mu = p[:, :Z]
        lv = p[:, Z:]
        kl = 0.5 * jnp.sum(
            jnp.exp(lv) + mu * mu - 1.0 - lv, axis=1, keepdims=True
        )
        eps_e = eps_ref[e]
        if best_kl is None:
            best_kl, best_mu, best_lv, best_eps = kl, mu, lv, eps_e
            widx = jnp.zeros_like(kl, dtype=jnp.int32)
        else:
            upd = kl > best_kl
            best_kl = jnp.where(upd, kl, best_kl)
            best_mu = jnp.where(upd, mu, best_mu)
            best_lv = jnp.where(upd, lv, best_lv)
            best_eps = jnp.where(upd, eps_e, best_eps)
            widx = jnp.where(upd, e, widx)
    z = best_mu + best_eps * jnp.exp(0.5 * best_lv)
    pieces = [jnp.where(widx == e, z, 0.0) for e in range(E)]
    z_cat = jnp.concatenate(pieces, axis=1)
    delta = jnp.dot(z_cat, wc_cat_ref[...], preferred_element_type=jnp.float32)
    oh = jnp.where(
        widx == jax.lax.broadcasted_iota(jnp.int32, (BN, Z), 1), 1.0, 0.0
    )
    delta = delta + jnp.dot(oh, bc_pad_ref[...], preferred_element_type=jnp.float32)
    return h + delta


def _fused_kernel(
    x_ref,
    wre_ref, bre_ref, wce_ref, bce_ref,
    wr_ref, br_ref, wc_ref, bc_ref,
    wrd_ref, brd_ref, wcd_ref, bcd_ref,
    ee_ref, e1_ref, e2_ref, ed_ref,
    out_ref,
):
    h = x_ref[...]
    h = _single_stage(h, wre_ref[0], bre_ref[...], wce_ref[0], bce_ref[...], ee_ref[0])
    h = _expert_stage(h, wr_ref, br_ref, wc_ref, bc_ref, e1_ref)
    h = _expert_stage(h, wr_ref, br_ref, wc_ref, bc_ref, e2_ref)
    h = _single_stage(h, wrd_ref[0], brd_ref[...], wcd_ref[0], bcd_ref[...], ed_ref[0])
    out_ref[...] = h


def _const_spec(shape):
    nd = len(shape)
    return pl.BlockSpec(shape, lambda i: (0,) * nd)


@jax.jit
def kernel(
    x,
    W_rec_enc, b_rec_enc, W_comp_enc, b_comp_enc,
    W_rec, b_rec, W_comp, b_comp,
    W_rec_dec, b_rec_dec, W_comp_dec, b_comp_dec,
    eps_enc, eps_c1, eps_c2, eps_dec,
):
    wc_cat = W_comp.reshape(E * Z, D)
    bc_pad = jnp.pad(b_comp, ((0, Z - E), (0, 0)))
    grid = (N // BN,)
    return pl.pallas_call(
        _fused_kernel,
        grid=grid,
        in_specs=[
            pl.BlockSpec((BN, D), lambda i: (i, 0)),
            _const_spec((1, D, 2 * Z)),
            _const_spec((1, 2 * Z)),
            _const_spec((1, Z, D)),
            _const_spec((1, D)),
            _const_spec((E, D, 2 * Z)),
            _const_spec((E, 2 * Z)),
            _const_spec((E * Z, D)),
            _const_spec((Z, D)),
            _const_spec((1, D, 2 * Z)),
            _const_spec((1, 2 * Z)),
            _const_spec((1, Z, D)),
            _const_spec((1, D)),
            pl.BlockSpec((1, BN, Z), lambda i: (0, i, 0)),
            pl.BlockSpec((E, BN, Z), lambda i: (0, i, 0)),
            pl.BlockSpec((E, BN, Z), lambda i: (0, i, 0)),
            pl.BlockSpec((1, BN, Z), lambda i: (0, i, 0)),
        ],
        out_specs=pl.BlockSpec((BN, D), lambda i: (i, 0)),
        out_shape=jax.ShapeDtypeStruct((N, D), jnp.float32),
        compiler_params=pltpu.CompilerParams(
            dimension_semantics=("arbitrary",),
        ),
    )(
        x,
        W_rec_enc, b_rec_enc, W_comp_enc, b_comp_enc,
        W_rec, b_rec, wc_cat, bc_pad,
        W_rec_dec, b_rec_dec, W_comp_dec, b_comp_dec,
        eps_enc, eps_c1, eps_c2, eps_dec,
    )
